# baseline (speedup 1.0000x reference)
import functools

import jax
import jax.numpy as jnp
from jax.experimental import pallas as pl

_B, _N, _S, _K, _CH = 8, 4096, 1024, 24, 256


def _index_points(points, idx):
    b = points.shape[0]
    flat = idx.reshape(b, -1)
    out = jnp.take_along_axis(points, flat[:, :, None], axis=1)
    return out.reshape(idx.shape + (points.shape[-1],))


def _fps(xyz, npoint):
    b, n, _ = xyz.shape

    def body(i, state):
        centroids, distance, farthest = state
        centroids = centroids.at[:, i].set(farthest)
        centroid = jnp.take_along_axis(xyz, farthest[:, None, None], axis=1)
        dist = jnp.sum((xyz - centroid) ** 2, axis=-1)
        distance = jnp.minimum(distance, dist)
        farthest = jnp.argmax(distance, axis=-1).astype(jnp.int32)
        return centroids, distance, farthest

    centroids = jnp.zeros((b, npoint), dtype=jnp.int32)
    distance = jnp.full((b, n), 1e10, dtype=xyz.dtype)
    farthest = jnp.zeros((b,), dtype=jnp.int32)
    centroids, _, _ = jax.lax.fori_loop(0, npoint, body, (centroids, distance, farthest))
    return centroids


def _knn(k, xyz, new_xyz):
    sq = (jnp.sum(new_xyz ** 2, axis=-1)[:, :, None]
          + jnp.sum(xyz ** 2, axis=-1)[:, None, :]
          - 2.0 * jnp.einsum('bsc,bnc->bsn', new_xyz, xyz))
    _, idx = jax.lax.top_k(-sq, k)
    return idx


def _identity_body(x_ref, o_ref):
    o_ref[...] = x_ref[...]


def kernel(xyz, points, affine_alpha, affine_beta):
    b = xyz.shape[0]
    fps_idx = _fps(jax.lax.stop_gradient(xyz), _S)
    new_xyz = _index_points(xyz, fps_idx)
    new_points = _index_points(points, fps_idx)
    idx = _knn(_K, jax.lax.stop_gradient(xyz), jax.lax.stop_gradient(new_xyz))
    grouped_xyz = _index_points(xyz, idx)
    grouped_points = _index_points(points, idx)
    grouped_points = jnp.concatenate([grouped_points, grouped_xyz], axis=-1)
    mean = jnp.concatenate([new_points, new_xyz], axis=-1)[:, :, None, :]
    diff = grouped_points - mean
    std = jnp.std(diff.reshape(b, -1), axis=-1, ddof=1, keepdims=True)[:, :, None, None]
    grouped_points = diff / (std + 1e-05)
    grouped_points = affine_alpha * grouped_points + affine_beta
    rep = jnp.broadcast_to(new_points[:, :, None, :], (b, _S, _K, new_points.shape[-1]))
    out = jnp.concatenate([grouped_points, rep], axis=-1)

    out2 = out.reshape(b * _S, _K * (2 * _CH + 3))
    out2 = pl.pallas_call(
        _identity_body,
        out_shape=jax.ShapeDtypeStruct(out2.shape, out2.dtype),
        grid=(b * _S // 128,),
        in_specs=[pl.BlockSpec((128, out2.shape[1]), lambda i: (i, 0))],
        out_specs=pl.BlockSpec((128, out2.shape[1]), lambda i: (i, 0)),
    )(out2)
    out = out2.reshape(b, _S, _K, 2 * _CH + 3)
    return new_xyz, out


# speedup vs baseline: 1.6169x; 1.6169x over previous
import functools

import jax
import jax.numpy as jnp
from jax.experimental import pallas as pl

_B, _N, _S, _K, _CH = 8, 4096, 1024, 24, 256


def _index_points(points, idx):
    b = points.shape[0]
    flat = idx.reshape(b, -1)
    out = jnp.take_along_axis(points, flat[:, :, None], axis=1)
    return out.reshape(idx.shape + (points.shape[-1],))


def _fps_body(xt_ref, idx_ref, dist_ref, far_ref):
    _, b, n = xt_ref.shape
    i = pl.program_id(0)

    @pl.when(i == 0)
    def _init():
        dist_ref[...] = jnp.full((b, n), 1e10, jnp.float32)
        far_ref[...] = jnp.zeros((b, 128), jnp.int32)

    far = far_ref[:, :1]
    idx_ref[...] = far.reshape(1, b, 1)

    x = xt_ref[0]
    y = xt_ref[1]
    z = xt_ref[2]
    iota_n = jax.lax.broadcasted_iota(jnp.int32, (b, n), 1)
    onehot = iota_n == far
    zero = jnp.zeros((b, n), jnp.float32)
    cx = jnp.sum(jnp.where(onehot, x, zero), axis=1, keepdims=True)
    cy = jnp.sum(jnp.where(onehot, y, zero), axis=1, keepdims=True)
    cz = jnp.sum(jnp.where(onehot, z, zero), axis=1, keepdims=True)
    d = (x - cx) ** 2 + (y - cy) ** 2 + (z - cz) ** 2
    dist = jnp.minimum(dist_ref[...], d)
    dist_ref[...] = dist
    m = jnp.max(dist, axis=1, keepdims=True)
    new_far = jnp.min(jnp.where(dist == m, iota_n, n), axis=1, keepdims=True)
    far_ref[:, :1] = new_far


def _fps(xyz, npoint):
    b, n, _ = xyz.shape
    xt = jnp.transpose(xyz, (2, 0, 1))
    from jax.experimental.pallas import tpu as pltpu
    idx = pl.pallas_call(
        _fps_body,
        grid=(npoint,),
        in_specs=[pl.BlockSpec((3, b, n), lambda i: (0, 0, 0))],
        out_specs=pl.BlockSpec((1, b, 1), lambda i: (i, 0, 0)),
        out_shape=jax.ShapeDtypeStruct((npoint, b, 1), jnp.int32),
        scratch_shapes=[pltpu.VMEM((b, n), jnp.float32),
                        pltpu.VMEM((b, 128), jnp.int32)],
    )(xt)
    return jnp.transpose(idx.reshape(npoint, b), (1, 0))


def _knn(k, xyz, new_xyz):
    sq = (jnp.sum(new_xyz ** 2, axis=-1)[:, :, None]
          + jnp.sum(xyz ** 2, axis=-1)[:, None, :]
          - 2.0 * jnp.einsum('bsc,bnc->bsn', new_xyz, xyz))
    _, idx = jax.lax.top_k(-sq, k)
    return idx


def _identity_body(x_ref, o_ref):
    o_ref[...] = x_ref[...]


def kernel(xyz, points, affine_alpha, affine_beta):
    b = xyz.shape[0]
    fps_idx = _fps(jax.lax.stop_gradient(xyz), _S)
    new_xyz = _index_points(xyz, fps_idx)
    new_points = _index_points(points, fps_idx)
    idx = _knn(_K, jax.lax.stop_gradient(xyz), jax.lax.stop_gradient(new_xyz))
    grouped_xyz = _index_points(xyz, idx)
    grouped_points = _index_points(points, idx)
    grouped_points = jnp.concatenate([grouped_points, grouped_xyz], axis=-1)
    mean = jnp.concatenate([new_points, new_xyz], axis=-1)[:, :, None, :]
    diff = grouped_points - mean
    std = jnp.std(diff.reshape(b, -1), axis=-1, ddof=1, keepdims=True)[:, :, None, None]
    grouped_points = diff / (std + 1e-05)
    grouped_points = affine_alpha * grouped_points + affine_beta
    rep = jnp.broadcast_to(new_points[:, :, None, :], (b, _S, _K, new_points.shape[-1]))
    out = jnp.concatenate([grouped_points, rep], axis=-1)

    out2 = out.reshape(b * _S, _K * (2 * _CH + 3))
    out2 = pl.pallas_call(
        _identity_body,
        out_shape=jax.ShapeDtypeStruct(out2.shape, out2.dtype),
        grid=(b * _S // 128,),
        in_specs=[pl.BlockSpec((128, out2.shape[1]), lambda i: (i, 0))],
        out_specs=pl.BlockSpec((128, out2.shape[1]), lambda i: (i, 0)),
    )(out2)
    out = out2.reshape(b, _S, _K, 2 * _CH + 3)
    return new_xyz, out


# speedup vs baseline: 8.0185x; 4.9590x over previous
import functools

import jax
import jax.numpy as jnp
from jax import lax
from jax.experimental import pallas as pl
from jax.experimental.pallas import tpu as pltpu
from jax.experimental.pallas import tpu_sc as plsc

_B, _N, _S, _K, _CH = 8, 4096, 1024, 24, 256


def _index_points(points, idx):
    b = points.shape[0]
    flat = idx.reshape(b, -1)
    out = jnp.take_along_axis(points, flat[:, :, None], axis=1)
    return out.reshape(idx.shape + (points.shape[-1],))


def _fps_body(xt_ref, idx_ref, dist_ref, far_ref):
    _, b, n = xt_ref.shape
    i = pl.program_id(0)

    @pl.when(i == 0)
    def _init():
        dist_ref[...] = jnp.full((b, n), 1e10, jnp.float32)
        far_ref[...] = jnp.zeros((b, 128), jnp.int32)

    far = far_ref[:, :1]
    idx_ref[...] = far.reshape(1, b, 1)

    x = xt_ref[0]
    y = xt_ref[1]
    z = xt_ref[2]
    iota_n = jax.lax.broadcasted_iota(jnp.int32, (b, n), 1)
    onehot = iota_n == far
    zero = jnp.zeros((b, n), jnp.float32)
    cx = jnp.sum(jnp.where(onehot, x, zero), axis=1, keepdims=True)
    cy = jnp.sum(jnp.where(onehot, y, zero), axis=1, keepdims=True)
    cz = jnp.sum(jnp.where(onehot, z, zero), axis=1, keepdims=True)
    d = (x - cx) ** 2 + (y - cy) ** 2 + (z - cz) ** 2
    dist = jnp.minimum(dist_ref[...], d)
    dist_ref[...] = dist
    m = jnp.max(dist, axis=1, keepdims=True)
    new_far = jnp.min(jnp.where(dist == m, iota_n, n), axis=1, keepdims=True)
    far_ref[:, :1] = new_far


def _fps(xyz, npoint):
    b, n, _ = xyz.shape
    xt = jnp.transpose(xyz, (2, 0, 1))
    idx = pl.pallas_call(
        _fps_body,
        grid=(npoint,),
        in_specs=[pl.BlockSpec((3, b, n), lambda i: (0, 0, 0))],
        out_specs=pl.BlockSpec((1, b, 1), lambda i: (i, 0, 0)),
        out_shape=jax.ShapeDtypeStruct((npoint, b, 1), jnp.int32),
        scratch_shapes=[pltpu.VMEM((b, n), jnp.float32),
                        pltpu.VMEM((b, 128), jnp.int32)],
    )(xt)
    return jnp.transpose(idx.reshape(npoint, b), (1, 0))


def _knn_body(snew_ref, xt_ref, idx_ref, *, k):
    _, sb, _ = snew_ref.shape
    _, _, n = xt_ref.shape
    snew = snew_ref[0]
    xn = xt_ref[0]
    sx, sy, sz = snew[:, 0:1], snew[:, 1:2], snew[:, 2:3]
    nx, ny, nz = xn[0:1, :], xn[1:2, :], xn[2:3, :]
    ss = sx * sx + sy * sy + sz * sz
    nn = nx * nx + ny * ny + nz * nz
    bf = lambda v: v.astype(jnp.bfloat16).astype(jnp.float32)
    cross = bf(sx) * bf(nx) + bf(sy) * bf(ny) + bf(sz) * bf(nz)
    d = ss + nn - 2.0 * cross
    iota_n = jax.lax.broadcasted_iota(jnp.int32, (sb, n), 1)
    wins = []
    for _ in range(k):
        m = jnp.min(d, axis=1, keepdims=True)
        cand = jnp.where(d == m, iota_n, n)
        win = jnp.min(cand, axis=1, keepdims=True)
        wins.append(win)
        d = jnp.where(cand == win, jnp.inf, d)
    idx_ref[...] = jnp.concatenate(wins, axis=1).reshape(1, sb, k)


def _knn(k, xyz, new_xyz):
    b, n, _ = xyz.shape
    s = new_xyz.shape[1]
    sb = 256 if s % 256 == 0 else s
    xt = jnp.transpose(xyz, (0, 2, 1))
    return pl.pallas_call(
        functools.partial(_knn_body, k=k),
        grid=(b, s // sb),
        in_specs=[
            pl.BlockSpec((1, sb, 3), lambda i, j: (i, j, 0)),
            pl.BlockSpec((1, 3, n), lambda i, j: (i, 0, 0)),
        ],
        out_specs=pl.BlockSpec((1, sb, k), lambda i, j: (i, j, 0)),
        out_shape=jax.ShapeDtypeStruct((b, s, k), jnp.int32),
    )(new_xyz, xt)


_OUTW = 2 * _CH + 3
_PADW = 528
_XW = 128
_NW = 32
_SPW = (_B * _S) // _NW
_G = 8
_CT = _CH + 16


def _sc_wid():
    return lax.axis_index("s") * 2 + lax.axis_index("c")


def _sc_base(wid):
    return (wid // 4) * _S + (wid % 4) * _SPW


def _sc_gather_group(t_hbm, xt_hbm, idxg_hbm, fpsg_hbm, g0, cidx_v, idx_v,
                     crows_v, nrows_v, cxyz_v, nxyz_v, sem):
    pltpu.sync_copy(fpsg_hbm.at[pl.ds(g0, _G)], cidx_v)
    pltpu.sync_copy(idxg_hbm.at[pl.ds(g0 * _K, _G * _K)], idx_v)
    cps = [
        pltpu.async_copy(t_hbm.at[cidx_v], crows_v, sem),
        pltpu.async_copy(xt_hbm.at[cidx_v], cxyz_v, sem),
        pltpu.async_copy(t_hbm.at[idx_v.at[pl.ds(0, 96)]],
                         nrows_v.at[pl.ds(0, 96), :], sem),
        pltpu.async_copy(t_hbm.at[idx_v.at[pl.ds(96, 96)]],
                         nrows_v.at[pl.ds(96, 96), :], sem),
        pltpu.async_copy(xt_hbm.at[idx_v.at[pl.ds(0, 96)]],
                         nxyz_v.at[pl.ds(0, 96), :], sem),
        pltpu.async_copy(xt_hbm.at[idx_v.at[pl.ds(96, 96)]],
                         nxyz_v.at[pl.ds(96, 96), :], sem),
    ]
    for c in cps:
        c.wait()


def _sc_stats_body(t_hbm, xt_hbm, idxg_hbm, fpsg_hbm, out_hbm,
                   cidx_v, idx_v, crows_v, nrows_v, cxyz_v, nxyz_v, acc_v,
                   sem):
    wid = _sc_wid()
    base = _sc_base(wid)

    def group(g, carry):
        g0 = base + g * _G
        _sc_gather_group(t_hbm, xt_hbm, idxg_hbm, fpsg_hbm, g0, cidx_v,
                         idx_v, crows_v, nrows_v, cxyz_v, nxyz_v, sem)
        as0, aq0 = carry
        for si in range(_G):
            crow = [crows_v[si, pl.ds(c * 16, 16)] for c in range(16)]
            cxy = cxyz_v[si, pl.ds(0, 16)]

            def kbody(kk, carry3, si=si, crow=crow, cxy=cxy):
                a1, a2 = carry3
                r = si * _K + kk
                for c in range(16):
                    dd = nrows_v[r, pl.ds(c * 16, 16)] - crow[c]
                    a1 = a1 + dd
                    a2 = a2 + dd * dd
                dd = nxyz_v[r, pl.ds(0, 16)] - cxy
                a1 = a1 + dd
                a2 = a2 + dd * dd
                return a1, a2

            as0, aq0 = lax.fori_loop(0, _K, kbody, (as0, aq0))
        return as0, aq0

    z = jnp.zeros((16,), jnp.float32)
    as0, aq0 = lax.fori_loop(0, _SPW // _G, group, (z, z))
    acc_v[0, :] = as0
    acc_v[1, :] = aq0
    pltpu.sync_copy(acc_v, out_hbm.at[wid])


def _sc_write_body(t_hbm, xt_hbm, idxg_hbm, fpsg_hbm, scale_hbm, bias_hbm,
                   out_hbm, cidx_v, idx_v, crows_v, nrows_v, cxyz_v, nxyz_v,
                   scale_hv, bias_hv, stage_v, sem):
    wid = _sc_wid()
    base = _sc_base(wid)
    b = wid // 4
    pltpu.sync_copy(scale_hbm.at[b], scale_hv)
    pltpu.sync_copy(bias_hbm.at[b], bias_hv)
    scal = [scale_hv[pl.ds(c * 16, 16)] for c in range(16)]
    bia = [bias_hv[pl.ds(c * 16, 16)] for c in range(16)]
    sxy = scale_hv[pl.ds(_CH, 16)]
    bxy = bias_hv[pl.ds(_CH, 16)]

    def group(g, gcarry):
        g0 = base + g * _G
        _sc_gather_group(t_hbm, xt_hbm, idxg_hbm, fpsg_hbm, g0, cidx_v,
                         idx_v, crows_v, nrows_v, cxyz_v, nxyz_v, sem)
        for si in range(_G):
            crow = [crows_v[si, pl.ds(c * 16, 16)] for c in range(16)]
            cxy = cxyz_v[si, pl.ds(0, 16)]

            def kbody(kk, kcarry, si=si, crow=crow, cxy=cxy):
                r = si * _K + kk
                for c in range(16):
                    v = (nrows_v[r, pl.ds(c * 16, 16)] - crow[c]) * scal[c] + bia[c]
                    stage_v[kk, pl.ds(c * 16, 16)] = v
                for c in range(16):
                    stage_v[kk, pl.ds(_CH + c * 16, 16)] = crow[c]
                v = (nxyz_v[r, pl.ds(0, 16)] - cxy) * sxy + bxy
                stage_v[kk, pl.ds(2 * _CH, 16)] = v
                return kcarry

            lax.fori_loop(0, _K, kbody, 0)
            pltpu.sync_copy(stage_v,
                            out_hbm.at[pl.ds((g0 + si) * _K, _K), :])
        return gcarry

    lax.fori_loop(0, _SPW // _G, group, 0)


def _sc_mesh():
    return plsc.VectorSubcoreMesh(core_axis_name="c", subcore_axis_name="s")


def _sc_stats(t2d, xt2d, idxg, fpsg):
    kfn = pl.kernel(
        _sc_stats_body,
        mesh=_sc_mesh(),
        out_type=jax.ShapeDtypeStruct((_NW, 2, 16), jnp.float32),
        scratch_types=[
            pltpu.VMEM((_G,), jnp.int32),
            pltpu.VMEM((_G * _K,), jnp.int32),
            pltpu.VMEM((_G, _CH), jnp.float32),
            pltpu.VMEM((_G * _K, _CH), jnp.float32),
            pltpu.VMEM((_G, _XW), jnp.float32),
            pltpu.VMEM((_G * _K, _XW), jnp.float32),
            pltpu.VMEM((2, 16), jnp.float32),
            pltpu.SemaphoreType.DMA,
        ],
    )
    return kfn(t2d, xt2d, idxg, fpsg)


def _sc_write(t2d, xt2d, idxg, fpsg, scale_t, bias_t):
    kfn = pl.kernel(
        _sc_write_body,
        mesh=_sc_mesh(),
        out_type=jax.ShapeDtypeStruct((_B * _S * _K, _PADW), jnp.float32),
        scratch_types=[
            pltpu.VMEM((_G,), jnp.int32),
            pltpu.VMEM((_G * _K,), jnp.int32),
            pltpu.VMEM((_G, _CH), jnp.float32),
            pltpu.VMEM((_G * _K, _CH), jnp.float32),
            pltpu.VMEM((_G, _XW), jnp.float32),
            pltpu.VMEM((_G * _K, _XW), jnp.float32),
            pltpu.VMEM((_CT,), jnp.float32),
            pltpu.VMEM((_CT,), jnp.float32),
            pltpu.VMEM((_K, _PADW), jnp.float32),
            pltpu.SemaphoreType.DMA,
        ],
    )
    return kfn(t2d, xt2d, idxg, fpsg, scale_t, bias_t)


def _compact_body(x_ref, o_ref):
    x = x_ref[...]
    o_ref[...] = jnp.concatenate(
        [x[:, :_CH], x[:, 2 * _CH:2 * _CH + 3], x[:, _CH:2 * _CH]], axis=1)


def _compact(padded):
    rows = padded.shape[0]
    rb = 512
    return pl.pallas_call(
        _compact_body,
        grid=(rows // rb,),
        in_specs=[pl.BlockSpec((rb, _PADW), lambda i: (i, 0))],
        out_specs=pl.BlockSpec((rb, _OUTW), lambda i: (i, 0)),
        out_shape=jax.ShapeDtypeStruct((rows, _OUTW), jnp.float32),
    )(padded)


def kernel(xyz, points, affine_alpha, affine_beta):
    fps_idx = _fps(xyz, _S)
    new_xyz = _index_points(xyz, fps_idx)
    idx = _knn(_K, xyz, new_xyz)

    t2d = points.reshape(_B * _N, _CH)
    xt2d = jnp.concatenate(
        [xyz, jnp.zeros((_B, _N, _XW - 3), jnp.float32)],
        axis=-1).reshape(_B * _N, _XW)
    boff = jnp.arange(_B, dtype=jnp.int32) * _N
    idxg = (idx + boff[:, None, None]).reshape(-1)
    fpsg = (fps_idx + boff[:, None]).reshape(-1)

    part = _sc_stats(t2d, xt2d, idxg, fpsg)
    sums = part.reshape(_B, 4, 2, 16).sum(axis=(1, 3))
    n_el = jnp.float32(_S * _K * (_CH + 3))
    mean = sums[:, 0] / n_el
    var = (sums[:, 1] - n_el * mean * mean) / (n_el - 1.0)
    std = jnp.sqrt(var)

    alpha = affine_alpha.reshape(-1)
    beta = affine_beta.reshape(-1)
    zpad = jnp.zeros((_CT - _CH - 3,), jnp.float32)
    alpha_p = jnp.concatenate([alpha, zpad])
    beta_p = jnp.concatenate([beta, zpad])
    scale_t = alpha_p[None, :] / (std[:, None] + 1e-5)
    bias_t = jnp.broadcast_to(beta_p[None, :], (_B, _CT))

    padded = _sc_write(t2d, xt2d, idxg, fpsg, scale_t, bias_t)
    out = _compact(padded).reshape(_B, _S, _K, _OUTW)
    return new_xyz, out
